# baseline (speedup 1.0000x reference)
import jax
import jax.numpy as jnp
from jax import lax
from jax.experimental import pallas as pl
from jax.experimental.pallas import tpu as pltpu

_BS, _N, _C = 8, 4096, 128
_HID, _OUT = 64, 16
_U = 16
_PB = 8
_G = _BS // _PB
_UP = _U * _PB


def _body(morb_ref, vals_ref, W1_ref, b1_ref, W2_ref, b2_ref,
          W3_ref, b3_ref, out_ref):
    g = pl.program_id(0)
    row_u = lax.broadcasted_iota(jnp.int32, (_U, _N), 0)
    aggs = []
    for p in range(_PB):
        morb = morb_ref[pl.ds(g * _PB + p, 1), :]
        oh = jnp.where(jnp.broadcast_to(morb, (_U, _N)) == row_u, 1.0, 0.0)
        aggs.append(lax.dot_general(oh, vals_ref[0, p], (((1,), (0,)), ((), ())),
                                    preferred_element_type=jnp.float32))
    agg = jnp.concatenate(aggs, axis=0)
    rowsum = jnp.sum(agg, axis=1, keepdims=True)
    empty = rowsum == 0.0
    h = jax.nn.relu(lax.dot_general(agg, W1_ref[...], (((1,), (1,)), ((), ())),
                                    preferred_element_type=jnp.float32)
                    + b1_ref[...])
    h = jax.nn.relu(lax.dot_general(h, W2_ref[...], (((1,), (0,)), ((), ())),
                                    preferred_element_type=jnp.float32)
                    + b2_ref[...])
    t = lax.dot_general(h, W3_ref[...], (((1,), (1,)), ((), ())),
                        preferred_element_type=jnp.float32) + b3_ref[...]
    t = jnp.where(empty, 0.0, t)
    colg = lax.broadcasted_iota(jnp.int32, (_PB, _UP), 1)
    rowg = lax.broadcasted_iota(jnp.int32, (_PB, _UP), 0)
    sel = (colg // _U == rowg).astype(jnp.float32)
    out_ref[0] = jnp.dot(sel, t, preferred_element_type=jnp.float32)


def kernel(coords, vals, mask, W1, b1, W2, b2, W3, b3):
    morb = jnp.where(mask, coords[:, :, 1, 1], jnp.int32(_U)).astype(jnp.int32)
    vals3 = vals.reshape(_G, _PB, _N, _C)
    W1t, W3t = W1.T, W3.T
    b1r = b1.reshape(1, _HID)
    b2r = b2.reshape(1, _HID)
    b3r = b3.reshape(1, _OUT)

    out = pl.pallas_call(
        _body,
        grid=(_G,),
        in_specs=[
            pl.BlockSpec((_BS, _N), lambda g: (0, 0)),
            pl.BlockSpec((1, _PB, _N, _C), lambda g: (g, 0, 0, 0)),
            pl.BlockSpec((_HID, _C), lambda g: (0, 0)),
            pl.BlockSpec((1, _HID), lambda g: (0, 0)),
            pl.BlockSpec((_HID, _HID), lambda g: (0, 0)),
            pl.BlockSpec((1, _HID), lambda g: (0, 0)),
            pl.BlockSpec((_OUT, _HID), lambda g: (0, 0)),
            pl.BlockSpec((1, _OUT), lambda g: (0, 0)),
        ],
        out_specs=pl.BlockSpec((1, _PB, _OUT), lambda g: (g, 0, 0)),
        out_shape=jax.ShapeDtypeStruct((_G, _PB, _OUT), jnp.float32),
    )(morb, vals3, W1t, b1r, W2, b2r, W3t, b3r)
    return out.reshape(_BS, _OUT)


# speedup vs baseline: 1.0637x; 1.0637x over previous
import jax
import jax.numpy as jnp
from jax import lax
from jax.experimental import pallas as pl
from jax.experimental.pallas import tpu as pltpu

_BS, _N, _C = 8, 4096, 128
_HID, _OUT = 64, 16
_U = 16
_PB = 4
_G = _BS // _PB
_UP = _U * _PB


def _body(morb_ref, vals_ref, W1_ref, b1_ref, W2_ref, b2_ref,
          W3_ref, b3_ref, out_ref):
    g = pl.program_id(0)
    row_u = lax.broadcasted_iota(jnp.int32, (_U, _N), 0)
    aggs = []
    for p in range(_PB):
        morb = morb_ref[pl.ds(g * _PB + p, 1), :]
        oh = jnp.where(jnp.broadcast_to(morb, (_U, _N)) == row_u, 1.0, 0.0)
        aggs.append(lax.dot_general(oh, vals_ref[0, p], (((1,), (0,)), ((), ())),
                                    preferred_element_type=jnp.float32))
    agg = jnp.concatenate(aggs, axis=0)
    rowsum = jnp.sum(agg, axis=1, keepdims=True)
    empty = rowsum == 0.0
    h = jax.nn.relu(lax.dot_general(agg, W1_ref[...], (((1,), (1,)), ((), ())),
                                    preferred_element_type=jnp.float32)
                    + b1_ref[...])
    h = jax.nn.relu(lax.dot_general(h, W2_ref[...], (((1,), (0,)), ((), ())),
                                    preferred_element_type=jnp.float32)
                    + b2_ref[...])
    t = lax.dot_general(h, W3_ref[...], (((1,), (1,)), ((), ())),
                        preferred_element_type=jnp.float32) + b3_ref[...]
    t = jnp.where(empty, 0.0, t)
    colg = lax.broadcasted_iota(jnp.int32, (_PB, _UP), 1)
    rowg = lax.broadcasted_iota(jnp.int32, (_PB, _UP), 0)
    sel = (colg // _U == rowg).astype(jnp.float32)
    out_ref[0] = jnp.dot(sel, t, preferred_element_type=jnp.float32)


def kernel(coords, vals, mask, W1, b1, W2, b2, W3, b3):
    morb = jnp.where(mask, coords[:, :, 1, 1], jnp.int32(_U)).astype(jnp.int32)
    vals3 = vals.reshape(_G, _PB, _N, _C)
    W1t, W3t = W1.T, W3.T
    b1r = b1.reshape(1, _HID)
    b2r = b2.reshape(1, _HID)
    b3r = b3.reshape(1, _OUT)

    out = pl.pallas_call(
        _body,
        grid=(_G,),
        in_specs=[
            pl.BlockSpec((_BS, _N), lambda g: (0, 0)),
            pl.BlockSpec((1, _PB, _N, _C), lambda g: (g, 0, 0, 0)),
            pl.BlockSpec((_HID, _C), lambda g: (0, 0)),
            pl.BlockSpec((1, _HID), lambda g: (0, 0)),
            pl.BlockSpec((_HID, _HID), lambda g: (0, 0)),
            pl.BlockSpec((1, _HID), lambda g: (0, 0)),
            pl.BlockSpec((_OUT, _HID), lambda g: (0, 0)),
            pl.BlockSpec((1, _OUT), lambda g: (0, 0)),
        ],
        out_specs=pl.BlockSpec((1, _PB, _OUT), lambda g: (g, 0, 0)),
        out_shape=jax.ShapeDtypeStruct((_G, _PB, _OUT), jnp.float32),
    )(morb, vals3, W1t, b1r, W2, b2r, W3t, b3r)
    return out.reshape(_BS, _OUT)
